# Initial kernel scaffold; baseline (speedup 1.0000x reference)
import jax
import jax.numpy as jnp
from jax.experimental import pallas as pl


def kernel(x, adj, W1, b1, W2, b2, W3, b3):
    raise NotImplementedError("write your pallas kernel here")


# speedup vs baseline: 1.0105x; 1.0105x over previous
import jax
import jax.numpy as jnp
from jax.experimental import pallas as pl
from jax.experimental.pallas import tpu as pltpu

N = 10000
BM = 400

_PARAMS = pltpu.CompilerParams(vmem_limit_bytes=100 * 1024 * 1024)


def _xw_kernel(x_ref, w_ref, o_ref):
    o_ref[...] = jnp.dot(x_ref[...], w_ref[...],
                         preferred_element_type=jnp.float32)


def _layer1_kernel(adj_ref, a_ref, b_ref, w_ref, c_ref, anext_ref):
    a = a_ref[...]
    colsum = jnp.sum(a, axis=0, keepdims=True)
    c = adj_ref[...] - 0.5
    cb = c.astype(jnp.bfloat16)
    c_ref[...] = cb
    acc = jnp.dot(cb, a.astype(jnp.bfloat16),
                  preferred_element_type=jnp.float32)
    h = jnp.maximum(acc + 0.5 * colsum + b_ref[...], 0.0)
    anext_ref[...] = jnp.dot(h, w_ref[...],
                             preferred_element_type=jnp.float32)


def _layer2_kernel(c_ref, a_ref, b_ref, w_ref, anext_ref):
    a = a_ref[...]
    colsum = jnp.sum(a, axis=0, keepdims=True)
    acc = jnp.dot(c_ref[...], a.astype(jnp.bfloat16),
                  preferred_element_type=jnp.float32)
    h = jnp.maximum(acc + 0.5 * colsum + b_ref[...], 0.0)
    anext_ref[...] = jnp.dot(h, w_ref[...],
                             preferred_element_type=jnp.float32)


def _layer3_kernel(c_ref, a_ref, b_ref, o_ref):
    a = a_ref[...]
    colsum = jnp.sum(a, axis=0, keepdims=True)
    acc = jnp.dot(c_ref[...], a.astype(jnp.bfloat16),
                  preferred_element_type=jnp.float32)
    o_ref[...] = acc + 0.5 * colsum + b_ref[...]


def _full(shape):
    return pl.BlockSpec(shape, lambda i: (0,) * len(shape))


def _rows(width, dtype_ignored=None):
    return pl.BlockSpec((BM, width), lambda i: (i, 0))


def kernel(x, adj, W1, b1, W2, b2, W3, b3):
    f32 = jnp.float32
    b1 = b1.reshape(1, -1)
    b2 = b2.reshape(1, -1)
    b3 = b3.reshape(1, -1)
    nh0, nh1, ncl = W1.shape[1], W2.shape[1], W3.shape[1]
    grid = (N // BM,)

    A1 = pl.pallas_call(
        _xw_kernel,
        out_shape=jax.ShapeDtypeStruct((N, nh0), f32),
    )(x, W1)

    C, A2 = pl.pallas_call(
        _layer1_kernel,
        grid=grid,
        in_specs=[
            _rows(N),
            _full((N, nh0)),
            _full((1, nh0)),
            _full((nh0, nh1)),
        ],
        out_specs=[_rows(N), _rows(nh1)],
        out_shape=[
            jax.ShapeDtypeStruct((N, N), jnp.bfloat16),
            jax.ShapeDtypeStruct((N, nh1), f32),
        ],
        compiler_params=_PARAMS,
    )(adj, A1, b1, W2)

    A3 = pl.pallas_call(
        _layer2_kernel,
        grid=grid,
        in_specs=[
            _rows(N),
            _full((N, nh1)),
            _full((1, nh1)),
            _full((nh1, ncl)),
        ],
        out_specs=_rows(ncl),
        out_shape=jax.ShapeDtypeStruct((N, ncl), f32),
        compiler_params=_PARAMS,
    )(C, A2, b2, W3)

    out = pl.pallas_call(
        _layer3_kernel,
        grid=grid,
        in_specs=[
            _rows(N),
            _full((N, ncl)),
            _full((1, ncl)),
        ],
        out_specs=_rows(ncl),
        out_shape=jax.ShapeDtypeStruct((N, ncl), f32),
        compiler_params=_PARAMS,
    )(C, A3, b3)
    return out
